# baseline (speedup 1.0000x reference)
import numpy as np
import jax
import jax.numpy as jnp
from jax.experimental import pallas as pl
from jax.experimental.pallas import tpu as pltpu


def _build_idx_table(n):
    idx = [[0, 1, 2], [3, 4, 5]]
    idx += [[6, 7, i] for i in range(8, n)]
    idx += [[0, 3, 6], [1, 4, 7]]
    idx += [[2, 5, i] for i in range(8, n)]
    return np.array(idx, dtype=np.int32)


def _merged_runs(flat):
    runs = []
    p = 0
    while p < len(flat):
        j0 = int(flat[p])
        L = 1
        while p + L < len(flat) and int(flat[p + L]) == j0 + L:
            L += 1
        runs.append((p, j0, L))
        p += L
    return runs


_N = 26
_P = 120
_FLAT_IDX = _build_idx_table(_N).reshape(-1)
_RUNS = _merged_runs(_FLAT_IDX)


def _copy_body(x_ref, o_ref):
    for p, j, L in _RUNS:
        o_ref[:, p:p + L, :] = x_ref[:, j:j + L, :]


def kernel(x):
    b, n, s, d = x.shape
    assert n == _N
    f = s * d
    x3 = x.reshape(b, n, f)
    bb = 8
    grid = (b // bb,)
    out3 = pl.pallas_call(
        _copy_body,
        grid=grid,
        in_specs=[pl.BlockSpec((bb, n, f), lambda i: (i, 0, 0))],
        out_specs=pl.BlockSpec((bb, _P, f), lambda i: (i, 0, 0)),
        out_shape=jax.ShapeDtypeStruct((b, _P, f), x.dtype),
    )(x3)
    return out3.reshape(b, _P // 3, 3, s, d)


# speedup vs baseline: 1.0950x; 1.0950x over previous
import functools

import numpy as np
import jax
import jax.numpy as jnp
from jax import lax
from jax.experimental import pallas as pl
from jax.experimental.pallas import tpu as pltpu
from jax.experimental.pallas import tpu_sc as plsc


def _build_idx_table(n):
    idx = [[0, 1, 2], [3, 4, 5]]
    idx += [[6, 7, i] for i in range(8, n)]
    idx += [[0, 3, 6], [1, 4, 7]]
    idx += [[2, 5, i] for i in range(8, n)]
    return np.array(idx, dtype=np.int32)


def _merged_runs(flat):
    runs = []
    p = 0
    while p < len(flat):
        j0 = int(flat[p])
        L = 1
        while p + L < len(flat) and int(flat[p + L]) == j0 + L:
            L += 1
        runs.append((p, j0, L))
        p += L
    return runs


_N = 26
_P = 120
_F = 640
_RUNS = _merged_runs(_build_idx_table(_N).reshape(-1))
_B = 1024
_NC, _NS = 2, 16
_NW = _NC * _NS
_BPW = _B // _NW
_ROW_IN = _N * _F
_ROW_OUT = _P * _F
_IN_BYTES = _ROW_IN * 4
_OUT_BYTES = _ROW_OUT * 4


_DRAIN_CHUNK = 15360


def _fire_writes(in_buf, slot, out_hbm, b, sem):
    for p, j, L in _RUNS:
        pltpu.make_async_copy(
            in_buf.at[slot, pl.ds(j * _F, L * _F)],
            out_hbm.at[b, pl.ds(p * _F, L * _F)],
            sem,
        ).start()


def _drain_writes(in_buf, out_hbm, sem):
    for _ in range(_ROW_OUT // _DRAIN_CHUNK):
        pltpu.make_async_copy(
            out_hbm.at[0, pl.ds(0, _DRAIN_CHUNK)],
            in_buf.at[0, pl.ds(0, _DRAIN_CHUNK)],
            sem,
        ).wait()


def _sc_body(x_hbm, out_hbm, in_buf, sem_in0, sem_in1, sem_out0, sem_out1):
    wid = lax.axis_index("s") * _NC + lax.axis_index("c")
    base = wid * _BPW
    pltpu.make_async_copy(x_hbm.at[base], in_buf.at[0], sem_in0).start()

    def pair(i, carry):
        a = base + 2 * i

        @pl.when(i > 0)
        def _():
            _drain_writes(in_buf, out_hbm, sem_out1)

        pltpu.make_async_copy(x_hbm.at[a + 1], in_buf.at[1], sem_in1).start()

        pltpu.make_async_copy(x_hbm.at[a], in_buf.at[0], sem_in0).wait()
        _fire_writes(in_buf, 0, out_hbm, a, sem_out0)
        _drain_writes(in_buf, out_hbm, sem_out0)

        @pl.when(i + 1 < _BPW // 2)
        def _():
            pltpu.make_async_copy(x_hbm.at[a + 2], in_buf.at[0], sem_in0).start()

        pltpu.make_async_copy(x_hbm.at[a + 1], in_buf.at[1], sem_in1).wait()
        _fire_writes(in_buf, 1, out_hbm, a + 1, sem_out1)

        return carry

    lax.fori_loop(0, _BPW // 2, pair, 0)
    _drain_writes(in_buf, out_hbm, sem_out1)


def kernel(x):
    b, n, s, d = x.shape
    assert n == _N and s * d == _F and b == _B
    x2 = x.reshape(b, n * s * d)
    mesh = plsc.VectorSubcoreMesh(
        core_axis_name="c", subcore_axis_name="s",
        num_cores=_NC, num_subcores=_NS,
    )
    run = functools.partial(
        pl.kernel,
        out_type=jax.ShapeDtypeStruct((b, _ROW_OUT), jnp.float32),
        mesh=mesh,
        scratch_types=[
            pltpu.VMEM((2, _ROW_IN), jnp.float32),
            pltpu.SemaphoreType.DMA,
            pltpu.SemaphoreType.DMA,
            pltpu.SemaphoreType.DMA,
            pltpu.SemaphoreType.DMA,
        ],
    )
    out2 = run(_sc_body)(x2)
    return out2.reshape(b, _P // 3, 3, s, d)


# speedup vs baseline: 1.0972x; 1.0020x over previous
import numpy as np
import jax
import jax.numpy as jnp
from jax import lax
from jax.experimental import pallas as pl
from jax.experimental.pallas import tpu as pltpu
from jax.experimental.pallas import tpu_sc as plsc


_N = 26
_P = 120
_F = 640
_B = 1024
_NC, _NS = 2, 16
_NW = _NC * _NS
_BPW = _B // _NW
_ROW_IN = _N * _F
_ROW_OUT = _P * _F
_K = 4
_D = 2
_DRAIN_CHUNK = 15360


def _src_row(p):
    r = lax.div(p, 3)
    k = p - 3 * r
    q = p - 60
    qd = lax.div(q, 3)
    j_mid = jnp.where(k == 0, 6, jnp.where(k == 1, 7, r + 6))
    j_cross = (q - 3 * qd) * 3 + qd
    j_tail = jnp.where(k == 0, 2, jnp.where(k == 1, 5, r - 14))
    return jnp.where(
        p < 6, p,
        jnp.where(p < 60, j_mid, jnp.where(p < 66, j_cross, j_tail)))


def _fire_batch(in_buf, slot, out_hbm, b, sem):
    def fire_one(p, carry):
        j = _src_row(p)
        src_off = pl.multiple_of(j * _F, 128)
        dst_off = pl.multiple_of(p * _F, 128)
        pltpu.make_async_copy(
            in_buf.at[slot, pl.ds(src_off, _F)],
            out_hbm.at[b, pl.ds(dst_off, _F)],
            sem,
        ).start()
        return carry

    lax.fori_loop(0, _P, fire_one, 0)


def _drain_batch(in_buf, out_hbm, sem):
    for _ in range(_ROW_OUT // _DRAIN_CHUNK):
        pltpu.make_async_copy(
            out_hbm.at[0, pl.ds(0, _DRAIN_CHUNK)],
            in_buf.at[0, pl.ds(0, _DRAIN_CHUNK)],
            sem,
        ).wait()


def _sc_body(x_hbm, out_hbm, in_buf, *sems):
    sem_in = sems[:_K]
    sem_out = sems[_K:]
    wid = lax.axis_index("s") * _NC + lax.axis_index("c")
    base = wid * _BPW
    for s in range(_D):
        pltpu.make_async_copy(x_hbm.at[base + s], in_buf.at[s], sem_in[s]).start()

    def group(g, carry):
        for s in range(_K):
            i = _K * g + s
            b = base + i
            sd = (s + _D) % _K

            @pl.when(jnp.logical_and(i >= _D, i < _BPW - _D))
            def _():
                _drain_batch(in_buf, out_hbm, sem_out[sd])

            @pl.when(i < _BPW - _D)
            def _():
                pltpu.make_async_copy(
                    x_hbm.at[b + _D], in_buf.at[sd], sem_in[sd]).start()

            pltpu.make_async_copy(x_hbm.at[b], in_buf.at[s], sem_in[s]).wait()
            _fire_batch(in_buf, s, out_hbm, b, sem_out[s])
        return carry

    lax.fori_loop(0, _BPW // _K, group, 0)
    for s in range(_K):
        _drain_batch(in_buf, out_hbm, sem_out[s])


def kernel(x):
    b, n, s, d = x.shape
    assert n == _N and s * d == _F and b == _B
    x2 = x.reshape(b, n * s * d)
    mesh = plsc.VectorSubcoreMesh(
        core_axis_name="c", subcore_axis_name="s",
        num_cores=_NC, num_subcores=_NS,
    )
    out2 = pl.kernel(
        _sc_body,
        out_type=jax.ShapeDtypeStruct((b, _ROW_OUT), jnp.float32),
        mesh=mesh,
        scratch_types=(
            [pltpu.VMEM((_K, _ROW_IN), jnp.float32)]
            + [pltpu.SemaphoreType.DMA] * (2 * _K)
        ),
    )(x2)
    return out2.reshape(b, _P // 3, 3, s, d)
